# baseline (speedup 1.0000x reference)
import functools

import jax
import jax.numpy as jnp
from jax import lax
from jax.experimental import pallas as pl
from jax.experimental.pallas import tpu as pltpu
from jax.experimental.pallas import tpu_sc as plsc

B = 16384
D = 32
V = 1000000
MAIN = 999936
TAIL = V - MAIN
L = 16
NC, NS = 2, 16
NW = NC * NS
BPW = B // NW
CHUNK = 128
NCHUNK = BPW // CHUNK
NGROUP = BPW // L
DCH = 35712
NDCH = MAIN // DCH


def _rsqrt16(s):
    i = plsc.bitcast(s, jnp.int32)
    i = jnp.int32(0x5F3759DF) - (i >> 1)
    y = plsc.bitcast(i, jnp.float32)
    half = s * jnp.float32(0.5)
    for _ in range(3):
        y = y * (jnp.float32(1.5) - half * y * y)
    return jnp.minimum(y, jnp.float32(1e12))


_mesh = plsc.VectorSubcoreMesh(core_axis_name="c", subcore_axis_name="s")


@functools.partial(
    pl.kernel,
    out_type=jax.ShapeDtypeStruct((D * V,), jnp.float32),
    mesh=_mesh,
    scratch_types=[
        pltpu.VMEM((DCH,), jnp.float32),
        pltpu.VMEM((DCH,), jnp.float32),
        pltpu.SemaphoreType.DMA,
        pltpu.SemaphoreType.DMA,
        pltpu.SemaphoreType.DMA,
    ],
    compiler_params=pltpu.CompilerParams(needs_layout_passes=False),
)
def _detile(tab3, flat_out, buf0, buf1, sem_in, sem_o0, sem_o1):
    wid = lax.axis_index("s") * NC + lax.axis_index("c")
    ct = wid // 8
    cs = wid - ct * 8
    bufs = [buf0, buf1]
    osem = [sem_o0, sem_o1]

    def start_in(k):
        pltpu.async_copy(
            tab3.at[ct, cs, pl.ds(k * DCH, DCH)], bufs[k % 2], sem_in
        )

    def wait_in(k):
        pltpu.make_async_copy(
            tab3.at[0, 0, pl.ds(0, DCH)], bufs[k % 2], sem_in
        ).wait()

    def start_out(k):
        pltpu.async_copy(
            bufs[k % 2],
            flat_out.at[pl.ds(wid * V + k * DCH, DCH)],
            osem[k % 2],
        )

    def wait_out(k):
        pltpu.make_async_copy(
            bufs[k % 2], flat_out.at[pl.ds(0, DCH)], osem[k % 2]
        ).wait()

    start_in(0)
    for k in range(NDCH):
        wait_in(k)
        if k + 1 < NDCH:
            if k + 1 >= 2:
                wait_out(k - 1)
            start_in(k + 1)
        start_out(k)
    wait_out(NDCH - 2)
    wait_out(NDCH - 1)


@functools.partial(
    pl.kernel,
    out_type=jax.ShapeDtypeStruct((D, B), jnp.float32),
    mesh=_mesh,
    scratch_types=[
        pltpu.VMEM((BPW,), jnp.int32),
        pltpu.VMEM((D, BPW), jnp.float32),
        pltpu.VMEM((TAIL, D), jnp.float32),
        pltpu.SemaphoreType.DMA,
    ],
    compiler_params=pltpu.CompilerParams(needs_layout_passes=False),
)
def _gather_l2norm(flat, idx_hbm, tail_hbm, out_t, idx_v, cols_v, tail_v, sem):
    wid = lax.axis_index("s") * NC + lax.axis_index("c")
    base = wid * BPW

    pltpu.sync_copy(idx_hbm.at[pl.ds(base, BPW)], idx_v)
    pltpu.sync_copy(tail_hbm, tail_v)

    copies = [[] for _ in range(NCHUNK)]
    for k in range(NCHUNK):
        for c in range(D):
            copies[k].append(
                pltpu.async_copy(
                    flat.at[pl.ds(c * V, V)].at[
                        idx_v.at[pl.ds(k * CHUNK, CHUNK)]
                    ],
                    cols_v.at[c, pl.ds(k * CHUNK, CHUNK)],
                    sem,
                )
            )

    def group(g, carry):
        s = g * L
        idx16 = idx_v[pl.ds(s, L)]
        m = idx16 >= MAIN

        @pl.when(jnp.any(m))
        def _():
            tr = jnp.maximum(idx16 - MAIN, 0)
            for c in range(D):
                t = plsc.load_gather(
                    tail_v, [tr, jnp.full((L,), c, jnp.int32)], mask=m
                )
                v = cols_v[c, pl.ds(s, L)]
                cols_v[c, pl.ds(s, L)] = jnp.where(m, t, v)

        acc = jnp.zeros((L,), jnp.float32)
        vs = []
        for c in range(D):
            v = cols_v[c, pl.ds(s, L)]
            vs.append(v)
            acc = acc + v * v
        rinv = _rsqrt16(acc)
        for c in range(D):
            cols_v[c, pl.ds(s, L)] = vs[c] * rinv
        return carry

    for k in range(NCHUNK):
        for cp in copies[k]:
            cp.wait()
        lax.fori_loop(k * (CHUNK // L), (k + 1) * (CHUNK // L), group, 0)

    pltpu.sync_copy(cols_v, out_t.at[:, pl.ds(base, BPW)])


def kernel(x, table):
    tab3 = table.T.reshape(4, 8, V)
    tail_block = lax.slice(table, (MAIN, 0), (V, D))
    flat = _detile(tab3)
    out_t = _gather_l2norm(flat, x.astype(jnp.int32), tail_block)
    return out_t.T


# speedup vs baseline: 1.0184x; 1.0184x over previous
import functools

import jax
import jax.numpy as jnp
from jax import lax
from jax.experimental import pallas as pl
from jax.experimental.pallas import tpu as pltpu
from jax.experimental.pallas import tpu_sc as plsc

B = 16384
D = 32
V = 1000000
MAIN = 999936
TAIL = V - MAIN
L = 16
NC, NS = 2, 16
NW = NC * NS
BPW = B // NW
CHUNK = 128
NCHUNK = BPW // CHUNK
NGROUP = BPW // L
DCH = 55552
NDCH = MAIN // DCH


def _rsqrt16(s):
    i = plsc.bitcast(s, jnp.int32)
    i = jnp.int32(0x5F3759DF) - (i >> 1)
    y = plsc.bitcast(i, jnp.float32)
    half = s * jnp.float32(0.5)
    for _ in range(3):
        y = y * (jnp.float32(1.5) - half * y * y)
    return jnp.minimum(y, jnp.float32(1e12))


_mesh = plsc.VectorSubcoreMesh(core_axis_name="c", subcore_axis_name="s")


@functools.partial(
    pl.kernel,
    out_type=jax.ShapeDtypeStruct((D * V,), jnp.float32),
    mesh=_mesh,
    scratch_types=[
        pltpu.VMEM((DCH,), jnp.float32),
        pltpu.VMEM((DCH,), jnp.float32),
        pltpu.SemaphoreType.DMA,
        pltpu.SemaphoreType.DMA,
        pltpu.SemaphoreType.DMA,
    ],
    compiler_params=pltpu.CompilerParams(needs_layout_passes=False),
)
def _detile(tab3, flat_out, buf0, buf1, sem_in, sem_o0, sem_o1):
    wid = lax.axis_index("s") * NC + lax.axis_index("c")
    ct = wid // 8
    cs = wid - ct * 8
    bufs = [buf0, buf1]
    osem = [sem_o0, sem_o1]

    def start_in(k):
        pltpu.async_copy(
            tab3.at[ct, cs, pl.ds(k * DCH, DCH)], bufs[k % 2], sem_in
        )

    def wait_in(k):
        pltpu.make_async_copy(
            tab3.at[0, 0, pl.ds(0, DCH)], bufs[k % 2], sem_in
        ).wait()

    def start_out(k):
        pltpu.async_copy(
            bufs[k % 2],
            flat_out.at[pl.ds(wid * V + k * DCH, DCH)],
            osem[k % 2],
        )

    def wait_out(k):
        pltpu.make_async_copy(
            bufs[k % 2], flat_out.at[pl.ds(0, DCH)], osem[k % 2]
        ).wait()

    start_in(0)
    for k in range(NDCH):
        wait_in(k)
        if k + 1 < NDCH:
            if k + 1 >= 2:
                wait_out(k - 1)
            start_in(k + 1)
        start_out(k)
    wait_out(NDCH - 2)
    wait_out(NDCH - 1)


@functools.partial(
    pl.kernel,
    out_type=jax.ShapeDtypeStruct((D, B), jnp.float32),
    mesh=_mesh,
    scratch_types=[
        pltpu.VMEM((BPW,), jnp.int32),
        pltpu.VMEM((D, BPW), jnp.float32),
        pltpu.VMEM((TAIL, D), jnp.float32),
        pltpu.SemaphoreType.DMA,
    ],
    compiler_params=pltpu.CompilerParams(needs_layout_passes=False),
)
def _gather_l2norm(flat, idx_hbm, tail_hbm, out_t, idx_v, cols_v, tail_v, sem):
    wid = lax.axis_index("s") * NC + lax.axis_index("c")
    base = wid * BPW

    pltpu.sync_copy(idx_hbm.at[pl.ds(base, BPW)], idx_v)
    pltpu.sync_copy(tail_hbm, tail_v)

    copies = []
    for c in range(D):
        fslice = flat.at[pl.ds(c * V, V)]
        for k in range(NCHUNK):
            copies.append(
                pltpu.async_copy(
                    fslice.at[idx_v.at[pl.ds(k * CHUNK, CHUNK)]],
                    cols_v.at[c, pl.ds(k * CHUNK, CHUNK)],
                    sem,
                )
            )
    for cp in copies:
        cp.wait()

    def group(g, carry):
        s = g * L
        idx16 = idx_v[pl.ds(s, L)]
        m = idx16 >= MAIN

        @pl.when(jnp.any(m))
        def _():
            tr = jnp.maximum(idx16 - MAIN, 0)
            for c in range(D):
                t = plsc.load_gather(
                    tail_v, [tr, jnp.full((L,), c, jnp.int32)], mask=m
                )
                v = cols_v[c, pl.ds(s, L)]
                cols_v[c, pl.ds(s, L)] = jnp.where(m, t, v)

        acc = jnp.zeros((L,), jnp.float32)
        vs = []
        for c in range(D):
            v = cols_v[c, pl.ds(s, L)]
            vs.append(v)
            acc = acc + v * v
        rinv = _rsqrt16(acc)
        for c in range(D):
            cols_v[c, pl.ds(s, L)] = vs[c] * rinv
        return carry

    lax.fori_loop(0, NGROUP, group, 0)

    pltpu.sync_copy(cols_v, out_t.at[:, pl.ds(base, BPW)])


def kernel(x, table):
    tab3 = table.T.reshape(4, 8, V)
    tail_block = lax.slice(table, (MAIN, 0), (V, D))
    flat = _detile(tab3)
    out_t = _gather_l2norm(flat, x.astype(jnp.int32), tail_block)
    return out_t.T
